# baseline (speedup 1.0000x reference)
import functools

import jax
import jax.numpy as jnp
from jax import lax
from jax.experimental import pallas as pl
from jax.experimental.pallas import tpu as pltpu
from jax.experimental.pallas import tpu_sc as plsc

N = 10000
D = 128
E = 320000
DE = 16
P = 128
BATCH = 16384
S = 10

NC = 2
NS = 16
NW = NC * NS

NP = 10240
ROWS = S * NP
CW = ROWS // NW
SUBN = NP // NW
NSUBN = S
KPW = NP // NW
SUBW = 64
CPP = KPW // SUBW
NSUBW = S * CPP

BW = BATCH // NW
SUBB = 256
NSUBB = BW // SUBB

_MESH = dict(mesh=plsc.VectorSubcoreMesh(core_axis_name="c", subcore_axis_name="s"))


def _tc_prep_body(feats, w_prep, we0_neigh, w_edge_prep, we0_self, wn0_neigh,
                  feats_p, fp_tab, b1c, a2c):
    fp = jnp.dot(feats[:], w_prep[:], preferred_element_type=jnp.float32)
    feats_p[:] = fp
    fp_tab[:] = jnp.dot(fp, we0_neigh[:], preferred_element_type=jnp.float32)
    b1c[:] = jnp.dot(w_edge_prep[:], we0_self[:], preferred_element_type=jnp.float32)
    a2c[:] = jnp.dot(w_edge_prep[:], wn0_neigh[:], preferred_element_type=jnp.float32)


_tc_prep = pl.pallas_call(
    _tc_prep_body,
    out_shape=[
        jax.ShapeDtypeStruct((N, P), jnp.float32),
        jax.ShapeDtypeStruct((N, P), jnp.float32),
        jax.ShapeDtypeStruct((DE, P), jnp.float32),
        jax.ShapeDtypeStruct((DE, P), jnp.float32),
    ],
)


def _sc_narrow_body(n2e_flat, edge_emb, adj_flat,
                    eg_out, a0_out, a1_out,
                    posb, p0b, p1b, eidx, egbuf, a0v, a1v,
                    s_e, s_eg, s_a0, s_a1, s_egw, s_a0w, s_a1w):
    wid = lax.axis_index("s") * NC + lax.axis_index("c")
    koff = wid * SUBN
    iota = lax.iota(jnp.int32, 16)
    nvec = SUBN // 16

    h_e, h_g, h_w = {}, {}, {}

    def stage_pos(t):
        b = t % 3
        for v in range(nvec):
            vec = (iota + (koff + 16 * v)) * S + t
            posb[b, pl.ds(16 * v, 16)] = jnp.minimum(vec, N * S - 1)
        h_e[t] = pltpu.async_copy(n2e_flat.at[posb.at[b]], eidx.at[b], s_e.at[b])

    def stage_gather(t):
        b = t % 2
        b3 = t % 3
        if t >= 2:
            for h in h_w.pop(t - 2):
                h.wait()
        h_e.pop(t).wait()
        for v in range(nvec):
            ev = eidx[b3, pl.ds(16 * v, 16)]
            p0b[b, pl.ds(16 * v, 16)] = ev * 2
            p1b[b, pl.ds(16 * v, 16)] = ev * 2 + 1
        h_g[t] = (
            pltpu.async_copy(edge_emb.at[eidx.at[b3]], egbuf.at[b], s_eg.at[b]),
            pltpu.async_copy(adj_flat.at[p0b.at[b]], a0v.at[b], s_a0.at[b]),
            pltpu.async_copy(adj_flat.at[p1b.at[b]], a1v.at[b], s_a1.at[b]),
        )

    def stage_write(t):
        b = t % 2
        hg = h_g.pop(t)
        hg[0].wait()
        hg[1].wait()
        hg[2].wait()
        flat = t * NP + koff
        h_w[t] = (
            pltpu.async_copy(egbuf.at[b], eg_out.at[t, pl.ds(koff, SUBN)], s_egw.at[b]),
            pltpu.async_copy(a0v.at[b], a0_out.at[pl.ds(flat, SUBN)], s_a0w.at[b]),
            pltpu.async_copy(a1v.at[b], a1_out.at[pl.ds(flat, SUBN)], s_a1w.at[b]),
        )

    stage_pos(0)
    for t in range(NSUBN + 2):
        if 0 <= t - 2:
            stage_write(t - 2)
        if 0 <= t - 1 < NSUBN:
            stage_gather(t - 1)
        if t + 1 <= NSUBN - 1:
            stage_pos(t + 1)
    for hs in h_w.values():
        for h in hs:
            h.wait()


_sc_narrow = functools.partial(
    pl.kernel,
    out_type=[
        jax.ShapeDtypeStruct((S, NP, DE), jnp.float32),
        jax.ShapeDtypeStruct((ROWS,), jnp.int32),
        jax.ShapeDtypeStruct((ROWS,), jnp.int32),
    ],
    scratch_types=[
        pltpu.VMEM((3, SUBN), jnp.int32),
        pltpu.VMEM((2, SUBN), jnp.int32),
        pltpu.VMEM((2, SUBN), jnp.int32),
        pltpu.VMEM((3, SUBN), jnp.int32),
        pltpu.VMEM((2, SUBN, DE), jnp.float32),
        pltpu.VMEM((2, SUBN), jnp.int32),
        pltpu.VMEM((2, SUBN), jnp.int32),
        pltpu.SemaphoreType.DMA((3,)),
        pltpu.SemaphoreType.DMA((2,)),
        pltpu.SemaphoreType.DMA((2,)),
        pltpu.SemaphoreType.DMA((2,)),
        pltpu.SemaphoreType.DMA((2,)),
        pltpu.SemaphoreType.DMA((2,)),
        pltpu.SemaphoreType.DMA((2,)),
    ],
    compiler_params=pltpu.CompilerParams(use_tc_tiling_on_sc=False),
    **_MESH,
)(_sc_narrow_body)


def _sc_wide_body(a0_all, a1_all, fp_tab,
                  fm0_out, fm1_out,
                  fp_sh, a0v0, a0v1, a1v0, a1v1, fbuf0, fbuf1,
                  s_i0, s_i1, s_f0, s_f1, s_w0, s_w1):
    wid = lax.axis_index("s") * NC + lax.axis_index("c")
    sid = lax.axis_index("s")

    @pl.when(sid == 0)
    def _stage():
        pltpu.sync_copy(fp_tab, fp_sh)

    plsc.subcore_barrier()

    a0v = (a0v0, a0v1)
    a1v = (a1v0, a1v1)

    def off(t):
        j, c = t // CPP, t % CPP
        return j * NP + wid * KPW + c * SUBW

    h_idx, h_g, h_w = {}, {}, {}
    h_idx[0] = (
        pltpu.async_copy(a0_all.at[pl.ds(off(0), SUBW)], a0v[0], s_i0.at[0]),
        pltpu.async_copy(a1_all.at[pl.ds(off(0), SUBW)], a1v[0], s_i1.at[0]),
    )
    for t in range(NSUBW + 1):
        if 0 <= t - 1:
            j = t - 1
            hg = h_g.pop(j)
            hg[0].wait()
            hg[1].wait()
            pj, pc = j // CPP, j % CPP
            ko = wid * KPW + pc * SUBW
            h_w[j] = (
                pltpu.async_copy(fbuf0.at[j % 2], fm0_out.at[pj, pl.ds(ko, SUBW)], s_w0.at[j % 2]),
                pltpu.async_copy(fbuf1.at[j % 2], fm1_out.at[pj, pl.ds(ko, SUBW)], s_w1.at[j % 2]),
            )
        if t < NSUBW:
            if t >= 2:
                for h in h_w.pop(t - 2):
                    h.wait()
            for h in h_idx.pop(t):
                h.wait()
            b = t % 2
            h_g[t] = (
                pltpu.async_copy(fp_sh.at[a0v[b]], fbuf0.at[b], s_f0.at[b]),
                pltpu.async_copy(fp_sh.at[a1v[b]], fbuf1.at[b], s_f1.at[b]),
            )
            if t + 1 < NSUBW:
                nb = (t + 1) % 2
                h_idx[t + 1] = (
                    pltpu.async_copy(a0_all.at[pl.ds(off(t + 1), SUBW)], a0v[nb], s_i0.at[nb]),
                    pltpu.async_copy(a1_all.at[pl.ds(off(t + 1), SUBW)], a1v[nb], s_i1.at[nb]),
                )
    for hs in h_w.values():
        for h in hs:
            h.wait()


_sc_wide = functools.partial(
    pl.kernel,
    out_type=[
        jax.ShapeDtypeStruct((S, NP, P), jnp.float32),
        jax.ShapeDtypeStruct((S, NP, P), jnp.float32),
    ],
    scratch_types=[
        pltpu.VMEM_SHARED((N, P), jnp.float32),
        pltpu.VMEM((SUBW,), jnp.int32),
        pltpu.VMEM((SUBW,), jnp.int32),
        pltpu.VMEM((SUBW,), jnp.int32),
        pltpu.VMEM((SUBW,), jnp.int32),
        pltpu.VMEM((2, SUBW, P), jnp.float32),
        pltpu.VMEM((2, SUBW, P), jnp.float32),
        pltpu.SemaphoreType.DMA((2,)),
        pltpu.SemaphoreType.DMA((2,)),
        pltpu.SemaphoreType.DMA((2,)),
        pltpu.SemaphoreType.DMA((2,)),
        pltpu.SemaphoreType.DMA((2,)),
        pltpu.SemaphoreType.DMA((2,)),
    ],
    **_MESH,
)(_sc_wide_body)


def _tc_main_body(eg3, fm0, fm1, fpb, b1c, a2c, wn0_self, wn1_self, wn1_neigh, y):
    em = eg3[0]
    for j in range(1, S):
        em = em + eg3[j]
    em = em * (1.0 / S)
    g0 = jnp.maximum(
        jnp.dot(fpb[:], wn0_self[:], preferred_element_type=jnp.float32)
        + jnp.dot(em, a2c[:], preferred_element_type=jnp.float32), 0.0)
    gm = jnp.zeros_like(g0)
    for j in range(S):
        ep = jnp.dot(eg3[j], b1c[:], preferred_element_type=jnp.float32)
        gm = gm + jnp.maximum(ep + 0.5 * (fm0[j] + fm1[j]), 0.0)
    gm = gm * (1.0 / S)
    h0 = jnp.maximum(
        jnp.dot(g0, wn1_self[:], preferred_element_type=jnp.float32)
        + jnp.dot(gm, wn1_neigh[:], preferred_element_type=jnp.float32), 0.0)
    y[:, :P] = g0
    y[:, P:] = h0


NB = 400
_tc_main = pl.pallas_call(
    _tc_main_body,
    grid=(N // NB,),
    in_specs=[
        pl.BlockSpec((S, NB, DE), lambda i: (0, i, 0)),
        pl.BlockSpec((S, NB, P), lambda i: (0, i, 0)),
        pl.BlockSpec((S, NB, P), lambda i: (0, i, 0)),
        pl.BlockSpec((NB, P), lambda i: (i, 0)),
        pl.BlockSpec((DE, P), lambda i: (0, 0)),
        pl.BlockSpec((DE, P), lambda i: (0, 0)),
        pl.BlockSpec((P, P), lambda i: (0, 0)),
        pl.BlockSpec((P, P), lambda i: (0, 0)),
        pl.BlockSpec((P, P), lambda i: (0, 0)),
    ],
    out_specs=pl.BlockSpec((NB, 2 * P), lambda i: (i, 0)),
    out_shape=jax.ShapeDtypeStruct((N, 2 * P), jnp.float32),
)


def _sc_batch_body(ids_, y, out, idxv, ybuf, sem):
    wid = lax.axis_index("s") * NC + lax.axis_index("c")
    base = wid * BW
    for i in range(NSUBB):
        off = base + i * SUBB
        pltpu.sync_copy(ids_.at[pl.ds(off, SUBB)], idxv)
        pltpu.async_copy(y.at[idxv], ybuf, sem).wait()
        pltpu.sync_copy(ybuf, out.at[pl.ds(off, SUBB)])


_sc_batch = functools.partial(
    pl.kernel,
    out_type=jax.ShapeDtypeStruct((BATCH, 2 * P), jnp.float32),
    scratch_types=[
        pltpu.VMEM((SUBB,), jnp.int32),
        pltpu.VMEM((SUBB, 2 * P), jnp.float32),
        pltpu.SemaphoreType.DMA,
    ],
    **_MESH,
)(_sc_batch_body)


def kernel(ids, feats, edge_emb, node2edge_idx, edge_node_adj,
           W_prep, W_edge_prep, Wn0_self, Wn0_neigh,
           We0_self, We0_neigh, Wn1_self, Wn1_neigh):
    feats_p, fp_tab, b1c, a2c = _tc_prep(
        feats, W_prep, We0_neigh, W_edge_prep, We0_self, Wn0_neigh)
    eg, a0_all, a1_all = _sc_narrow(node2edge_idx.reshape(-1),
                                    edge_emb, edge_node_adj.reshape(-1))
    fm0, fm1 = _sc_wide(a0_all, a1_all, fp_tab)
    y = _tc_main(eg, fm0, fm1,
                 feats_p, b1c, a2c, Wn0_self, Wn1_self, Wn1_neigh)
    out = _sc_batch(ids, y)
    return out[None]


# speedup vs baseline: 1.6644x; 1.6644x over previous
import functools

import jax
import jax.numpy as jnp
from jax import lax
from jax.experimental import pallas as pl
from jax.experimental.pallas import tpu as pltpu
from jax.experimental.pallas import tpu_sc as plsc

N = 10000
D = 128
E = 320000
DE = 16
P = 128
BATCH = 16384
S = 10

NC = 2
NS = 16
NW = NC * NS

NP = N
ROWS = S * NP
KPW = 320
SUBN = KPW
NSUBN = S
SUBW = 64
CPP = KPW // SUBW
NSUBW = S * CPP

BW = BATCH // NW
SUBB = 256
NSUBB = BW // SUBB

_MESH = dict(mesh=plsc.VectorSubcoreMesh(core_axis_name="c", subcore_axis_name="s"))


def _tc_prep_body(feats, w_prep, we0_neigh, w_edge_prep, we0_self, wn0_neigh,
                  feats_p, fp_tab, b1c, a2c):
    fp = jnp.dot(feats[:], w_prep[:], preferred_element_type=jnp.float32)
    feats_p[:] = fp
    fp_tab[:] = jnp.dot(fp, we0_neigh[:], preferred_element_type=jnp.float32)
    b1c[:] = jnp.dot(w_edge_prep[:], we0_self[:], preferred_element_type=jnp.float32)
    a2c[:] = jnp.dot(w_edge_prep[:], wn0_neigh[:], preferred_element_type=jnp.float32)


_tc_prep = pl.pallas_call(
    _tc_prep_body,
    out_shape=[
        jax.ShapeDtypeStruct((N, P), jnp.float32),
        jax.ShapeDtypeStruct((N, P), jnp.float32),
        jax.ShapeDtypeStruct((DE, P), jnp.float32),
        jax.ShapeDtypeStruct((DE, P), jnp.float32),
    ],
)


def _sc_narrow_body(e_all, edge_emb, adj0, adj1,
                    eg_out, a0_out, a1_out,
                    eidx, egbuf, a0v, a1v,
                    s_idx, s_eg, s_a0, s_a1, s_egw, s_a0w, s_a1w):
    wid = lax.axis_index("s") * NC + lax.axis_index("c")
    koff = jnp.where(wid < NW - 1, wid * KPW, N - KPW)

    def flat(j):
        return j * NP + koff

    h_idx, h_g, h_w = {}, {}, {}
    h_idx[0] = pltpu.async_copy(e_all.at[pl.ds(flat(0), SUBN)], eidx.at[0], s_idx.at[0])
    for t in range(NSUBN + 1):
        if 0 <= t - 1:
            j = t - 1
            hg = h_g.pop(j)
            hg[0].wait()
            hg[1].wait()
            hg[2].wait()
            h_w[j] = (
                pltpu.async_copy(egbuf.at[j % 2], eg_out.at[j, pl.ds(koff, SUBN)], s_egw.at[j % 2]),
                pltpu.async_copy(a0v.at[j % 2], a0_out.at[pl.ds(flat(j), SUBN)], s_a0w.at[j % 2]),
                pltpu.async_copy(a1v.at[j % 2], a1_out.at[pl.ds(flat(j), SUBN)], s_a1w.at[j % 2]),
            )
        if t < NSUBN:
            if t >= 2:
                for h in h_w.pop(t - 2):
                    h.wait()
            h_idx.pop(t).wait()
            b = t % 2
            h_g[t] = (
                pltpu.async_copy(edge_emb.at[eidx.at[b]], egbuf.at[b], s_eg.at[b]),
                pltpu.async_copy(adj0.at[eidx.at[b]], a0v.at[b], s_a0.at[b]),
                pltpu.async_copy(adj1.at[eidx.at[b]], a1v.at[b], s_a1.at[b]),
            )
            if t + 1 < NSUBN:
                h_idx[t + 1] = pltpu.async_copy(
                    e_all.at[pl.ds(flat(t + 1), SUBN)], eidx.at[(t + 1) % 2], s_idx.at[(t + 1) % 2])
    for hs in h_w.values():
        for h in hs:
            h.wait()


_sc_narrow = functools.partial(
    pl.kernel,
    out_type=[
        jax.ShapeDtypeStruct((S, NP, DE), jnp.float32),
        jax.ShapeDtypeStruct((ROWS,), jnp.int32),
        jax.ShapeDtypeStruct((ROWS,), jnp.int32),
    ],
    scratch_types=[
        pltpu.VMEM((2, SUBN), jnp.int32),
        pltpu.VMEM((2, SUBN, DE), jnp.float32),
        pltpu.VMEM((2, SUBN), jnp.int32),
        pltpu.VMEM((2, SUBN), jnp.int32),
        pltpu.SemaphoreType.DMA((2,)),
        pltpu.SemaphoreType.DMA((2,)),
        pltpu.SemaphoreType.DMA((2,)),
        pltpu.SemaphoreType.DMA((2,)),
        pltpu.SemaphoreType.DMA((2,)),
        pltpu.SemaphoreType.DMA((2,)),
        pltpu.SemaphoreType.DMA((2,)),
    ],
    compiler_params=pltpu.CompilerParams(use_tc_tiling_on_sc=False),
    **_MESH,
)(_sc_narrow_body)


def _sc_wide_body(a0_all, a1_all, fp_tab,
                  fm0_out, fm1_out,
                  fp_sh, a0v0, a0v1, a1v0, a1v1, fbuf0, fbuf1,
                  s_i0, s_i1, s_f0, s_f1, s_w0, s_w1):
    wid = lax.axis_index("s") * NC + lax.axis_index("c")
    koff = jnp.where(wid < NW - 1, wid * KPW, N - KPW)
    sid = lax.axis_index("s")

    @pl.when(sid == 0)
    def _stage():
        pltpu.sync_copy(fp_tab, fp_sh)

    plsc.subcore_barrier()

    a0v = (a0v0, a0v1)
    a1v = (a1v0, a1v1)

    def off(t):
        j, c = t // CPP, t % CPP
        return j * NP + koff + c * SUBW

    h_idx, h_g, h_w = {}, {}, {}
    h_idx[0] = (
        pltpu.async_copy(a0_all.at[pl.ds(off(0), SUBW)], a0v[0], s_i0.at[0]),
        pltpu.async_copy(a1_all.at[pl.ds(off(0), SUBW)], a1v[0], s_i1.at[0]),
    )
    for t in range(NSUBW + 1):
        if 0 <= t - 1:
            j = t - 1
            hg = h_g.pop(j)
            hg[0].wait()
            hg[1].wait()
            pj, pc = j // CPP, j % CPP
            ko = koff + pc * SUBW
            h_w[j] = (
                pltpu.async_copy(fbuf0.at[j % 2], fm0_out.at[pj, pl.ds(ko, SUBW)], s_w0.at[j % 2]),
                pltpu.async_copy(fbuf1.at[j % 2], fm1_out.at[pj, pl.ds(ko, SUBW)], s_w1.at[j % 2]),
            )
        if t < NSUBW:
            if t >= 2:
                for h in h_w.pop(t - 2):
                    h.wait()
            for h in h_idx.pop(t):
                h.wait()
            b = t % 2
            h_g[t] = (
                pltpu.async_copy(fp_sh.at[a0v[b]], fbuf0.at[b], s_f0.at[b]),
                pltpu.async_copy(fp_sh.at[a1v[b]], fbuf1.at[b], s_f1.at[b]),
            )
            if t + 1 < NSUBW:
                nb = (t + 1) % 2
                h_idx[t + 1] = (
                    pltpu.async_copy(a0_all.at[pl.ds(off(t + 1), SUBW)], a0v[nb], s_i0.at[nb]),
                    pltpu.async_copy(a1_all.at[pl.ds(off(t + 1), SUBW)], a1v[nb], s_i1.at[nb]),
                )
    for hs in h_w.values():
        for h in hs:
            h.wait()


_sc_wide = functools.partial(
    pl.kernel,
    out_type=[
        jax.ShapeDtypeStruct((S, NP, P), jnp.float32),
        jax.ShapeDtypeStruct((S, NP, P), jnp.float32),
    ],
    scratch_types=[
        pltpu.VMEM_SHARED((N, P), jnp.float32),
        pltpu.VMEM((SUBW,), jnp.int32),
        pltpu.VMEM((SUBW,), jnp.int32),
        pltpu.VMEM((SUBW,), jnp.int32),
        pltpu.VMEM((SUBW,), jnp.int32),
        pltpu.VMEM((2, SUBW, P), jnp.float32),
        pltpu.VMEM((2, SUBW, P), jnp.float32),
        pltpu.SemaphoreType.DMA((2,)),
        pltpu.SemaphoreType.DMA((2,)),
        pltpu.SemaphoreType.DMA((2,)),
        pltpu.SemaphoreType.DMA((2,)),
        pltpu.SemaphoreType.DMA((2,)),
        pltpu.SemaphoreType.DMA((2,)),
    ],
    **_MESH,
)(_sc_wide_body)


def _tc_main_body(eg3, fm0, fm1, fpb, b1c, a2c, wn0_self, wn1_self, wn1_neigh, y):
    em = eg3[0]
    for j in range(1, S):
        em = em + eg3[j]
    em = em * (1.0 / S)
    g0 = jnp.maximum(
        jnp.dot(fpb[:], wn0_self[:], preferred_element_type=jnp.float32)
        + jnp.dot(em, a2c[:], preferred_element_type=jnp.float32), 0.0)
    gm = jnp.zeros_like(g0)
    for j in range(S):
        ep = jnp.dot(eg3[j], b1c[:], preferred_element_type=jnp.float32)
        gm = gm + jnp.maximum(ep + 0.5 * (fm0[j] + fm1[j]), 0.0)
    gm = gm * (1.0 / S)
    h0 = jnp.maximum(
        jnp.dot(g0, wn1_self[:], preferred_element_type=jnp.float32)
        + jnp.dot(gm, wn1_neigh[:], preferred_element_type=jnp.float32), 0.0)
    y[:, :P] = g0
    y[:, P:] = h0


NB = 400
_tc_main = pl.pallas_call(
    _tc_main_body,
    grid=(N // NB,),
    in_specs=[
        pl.BlockSpec((S, NB, DE), lambda i: (0, i, 0)),
        pl.BlockSpec((S, NB, P), lambda i: (0, i, 0)),
        pl.BlockSpec((S, NB, P), lambda i: (0, i, 0)),
        pl.BlockSpec((NB, P), lambda i: (i, 0)),
        pl.BlockSpec((DE, P), lambda i: (0, 0)),
        pl.BlockSpec((DE, P), lambda i: (0, 0)),
        pl.BlockSpec((P, P), lambda i: (0, 0)),
        pl.BlockSpec((P, P), lambda i: (0, 0)),
        pl.BlockSpec((P, P), lambda i: (0, 0)),
    ],
    out_specs=pl.BlockSpec((NB, 2 * P), lambda i: (i, 0)),
    out_shape=jax.ShapeDtypeStruct((N, 2 * P), jnp.float32),
)


def _sc_batch_body(ids_, y, out, idxv, ybuf, sem):
    wid = lax.axis_index("s") * NC + lax.axis_index("c")
    base = wid * BW
    for i in range(NSUBB):
        off = base + i * SUBB
        pltpu.sync_copy(ids_.at[pl.ds(off, SUBB)], idxv)
        pltpu.async_copy(y.at[idxv], ybuf, sem).wait()
        pltpu.sync_copy(ybuf, out.at[pl.ds(off, SUBB)])


_sc_batch = functools.partial(
    pl.kernel,
    out_type=jax.ShapeDtypeStruct((BATCH, 2 * P), jnp.float32),
    scratch_types=[
        pltpu.VMEM((SUBB,), jnp.int32),
        pltpu.VMEM((SUBB, 2 * P), jnp.float32),
        pltpu.SemaphoreType.DMA,
    ],
    **_MESH,
)(_sc_batch_body)


def kernel(ids, feats, edge_emb, node2edge_idx, edge_node_adj,
           W_prep, W_edge_prep, Wn0_self, Wn0_neigh,
           We0_self, We0_neigh, Wn1_self, Wn1_neigh):
    feats_p, fp_tab, b1c, a2c = _tc_prep(
        feats, W_prep, We0_neigh, W_edge_prep, We0_self, Wn0_neigh)
    n2e_wide = jnp.pad(node2edge_idx, ((0, 0), (0, 128 - S)))
    e_all = n2e_wide.T[:S].reshape(-1)
    eg, a0_all, a1_all = _sc_narrow(e_all, edge_emb,
                                    edge_node_adj[:, 0], edge_node_adj[:, 1])
    fm0, fm1 = _sc_wide(a0_all, a1_all, fp_tab)
    y = _tc_main(eg, fm0, fm1,
                 feats_p, b1c, a2c, Wn0_self, Wn1_self, Wn1_neigh)
    out = _sc_batch(ids, y)
    return out[None]
